# baseline (speedup 1.0000x reference)
import functools

import jax
import jax.numpy as jnp
import numpy as np
from jax.experimental import pallas as pl

IN_C = 32
OUT_C = 16
NQ = 2048
RF = 0.05
TEMP = 1.0
EPS = 1e-8


def _erf(x):
    s = jnp.sign(x)
    a = jnp.abs(x)
    t = 1.0 / (1.0 + 0.3275911 * a)
    poly = t * (0.254829592 + t * (-0.284496736 + t * (1.421413741 + t * (-1.453152027 + t * 1.061405429))))
    return s * (1.0 - poly * jnp.exp(-a * a))


def _gelu(x):
    return 0.5 * x * (1.0 + _erf(x * 0.7071067811865476))


def _epilogue_body(out_ref, bias_ref, wm1_ref, bm1_ref, wm2_ref, bm2_ref, o_ref):
    o = out_ref[...]
    o = _gelu(o + bias_ref[...])
    h2 = _gelu(
        jnp.dot(o, wm1_ref[...].T, preferred_element_type=jnp.float32) + bm1_ref[...],
    )
    o2 = jnp.dot(h2, wm2_ref[...].T, preferred_element_type=jnp.float32) + bm2_ref[...]
    o_ref[...] = o2 + o


def kernel(x, pos, query_pos, qmw, qmo, W_lin, b_lin, W_l1, b_l1, W_l2, filt, bias_out, W_m1, b_m1, W_m2, b_m2, B):
    max_nbr = int(np.floor(RF * (pos.shape[0] - 1))) + 1
    dist = query_pos[:, None, :] - pos[None, :, :]
    dist = (dist + 0.5) % 1.0 - 0.5
    edist = jnp.sum(dist**2, axis=-1)
    _, ind = jax.lax.top_k(-edist, max_nbr)
    dist_s = jnp.take_along_axis(dist, ind[:, :, None], axis=1)
    edist_s = jnp.take_along_axis(edist, ind, axis=1)[..., None]
    edist_s = edist_s - jnp.min(edist_s, axis=-2, keepdims=True)
    edist_s = edist_s / (jnp.max(edist_s, axis=-2, keepdims=True) + EPS)
    k_dist = jax.nn.softmax(-edist_s / TEMP, axis=-2)
    proj = 2.0 * np.pi * (dist_s @ B)
    kf = jnp.concatenate([jnp.sin(proj), jnp.cos(proj)], axis=-1)
    h = (kf @ W_l1.T + b_l1) * qmw[:, None, :] + qmo[:, None, :]
    h = jax.nn.gelu(h, approximate=False)
    k = h @ W_l2.T + filt[None, None, :]
    k = k * k_dist
    k = k.reshape(NQ, max_nbr, IN_C, OUT_C)
    xl = x @ W_lin.T + b_lin
    xg = xl[:, :, ind, :]
    out = jnp.einsum("qkcd,btqkc->btqd", k, xg)

    bsz, tsz = x.shape[0], x.shape[1]
    out_flat = out.reshape(bsz * tsz * NQ, OUT_C)
    out_final = pl.pallas_call(
        _epilogue_body,
        out_shape=jax.ShapeDtypeStruct((bsz * tsz * NQ, OUT_C), jnp.float32),
        grid=(4,),
        in_specs=[
            pl.BlockSpec((bsz * tsz * NQ // 4, OUT_C), lambda i: (i, 0)),
            pl.BlockSpec((OUT_C,), lambda i: (0,)),
            pl.BlockSpec(W_m1.shape, lambda i: (0, 0)),
            pl.BlockSpec(b_m1.shape, lambda i: (0,)),
            pl.BlockSpec(W_m2.shape, lambda i: (0, 0)),
            pl.BlockSpec(b_m2.shape, lambda i: (0,)),
        ],
        out_specs=pl.BlockSpec((bsz * tsz * NQ // 4, OUT_C), lambda i: (i, 0)),
    )(out_flat, bias_out, W_m1, b_m1, W_m2, b_m2)
    return (out_final.reshape(bsz, tsz, NQ, OUT_C), query_pos)


# speedup vs baseline: 1.5745x; 1.5745x over previous
import jax
import jax.numpy as jnp
import numpy as np
from jax.experimental import pallas as pl

IN_C = 32
OUT_C = 16
NQ = 2048
RF = 0.05
TEMP = 1.0
EPS = 1e-8
QB = 64


def _erf(x):
    s = jnp.sign(x)
    a = jnp.abs(x)
    t = 1.0 / (1.0 + 0.3275911 * a)
    poly = t * (0.254829592 + t * (-0.284496736 + t * (1.421413741 + t * (-1.453152027 + t * 1.061405429))))
    return s * (1.0 - poly * jnp.exp(-a * a))


def _gelu(x):
    return 0.5 * x * (1.0 + _erf(x * 0.7071067811865476))


def _main_body(K, H, BT,
               dist_ref, kd_ref, qmw_ref, qmo_ref, xe_ref, S_ref,
               B_ref, wl1_ref, bl1_ref, wl2p_ref, filtp_ref, o_ref):
    f32 = jnp.float32
    d2 = dist_ref[...]
    proj = 2.0 * np.pi * jnp.dot(d2, B_ref[...], preferred_element_type=f32)
    kf = jnp.concatenate([jnp.sin(proj), jnp.cos(proj)], axis=1)

    h = jnp.dot(kf, wl1_ref[...].T, preferred_element_type=f32) + bl1_ref[...]
    h = _gelu(h * qmw_ref[...] + qmo_ref[...])

    kdv = kd_ref[...]
    xe = xe_ref[...]
    S = S_ref[...]
    for d in range(OUT_C):
        w_d = wl2p_ref[d]
        kk_d = (jnp.dot(h, w_d.T, preferred_element_type=f32) + filtp_ref[d]) * kdv
        t = xe * kk_d[:, None, :]
        s = jnp.sum(t, axis=2)
        o_ref[d] = jnp.dot(S, s, preferred_element_type=f32)


def _epilogue_body(out_ref, bias_ref, wm1_ref, bm1_ref, wm2_ref, bm2_ref, o_ref):
    o = _gelu(out_ref[...] + bias_ref[...])
    h2 = _gelu(jnp.dot(o, wm1_ref[...].T, preferred_element_type=jnp.float32) + bm1_ref[...])
    o2 = jnp.dot(h2, wm2_ref[...].T, preferred_element_type=jnp.float32) + bm2_ref[...]
    o_ref[...] = o2 + o


def kernel(x, pos, query_pos, qmw, qmo, W_lin, b_lin, W_l1, b_l1, W_l2, filt,
           bias_out, W_m1, b_m1, W_m2, b_m2, B):
    V = pos.shape[0]
    K = int(np.floor(RF * (V - 1))) + 1
    H = W_l1.shape[0]
    bsz, tsz = x.shape[0], x.shape[1]
    BT = bsz * tsz

    dist = query_pos[:, None, :] - pos[None, :, :]
    dist = (dist + 0.5) % 1.0 - 0.5
    edist = jnp.sum(dist**2, axis=-1)
    _, ind = jax.lax.top_k(-edist, K)
    dist_s = jnp.take_along_axis(dist, ind[:, :, None], axis=1)
    edist_s = jnp.take_along_axis(edist, ind, axis=1)
    e = edist_s - jnp.min(edist_s, axis=1, keepdims=True)
    e = e / (jnp.max(e, axis=1, keepdims=True) + EPS)
    kd = jax.nn.softmax(-e / TEMP, axis=1)

    dist2 = dist_s.reshape(NQ * K, 2)
    kd_e = kd.reshape(NQ * K, 1)
    qmw_e = jnp.repeat(qmw, K, axis=0)
    qmo_e = jnp.repeat(qmo, K, axis=0)
    xl = (x @ W_lin.T + b_lin).reshape(BT, V, IN_C)
    xe = jnp.transpose(xl[:, ind.reshape(-1), :], (1, 0, 2))

    S = jnp.asarray(np.kron(np.eye(QB, dtype=np.float32),
                            np.ones((1, K), dtype=np.float32)))
    W_l2p = jnp.transpose(W_l2.reshape(IN_C, OUT_C, H), (1, 0, 2))
    filt_p = filt.reshape(IN_C, OUT_C).T

    grid = (NQ // QB,)
    body = lambda *refs: _main_body(K, H, BT, *refs)
    out_c = pl.pallas_call(
        body,
        out_shape=jax.ShapeDtypeStruct((OUT_C, NQ, BT), jnp.float32),
        grid=grid,
        in_specs=[
            pl.BlockSpec((QB * K, 2), lambda i: (i, 0)),
            pl.BlockSpec((QB * K, 1), lambda i: (i, 0)),
            pl.BlockSpec((QB * K, H), lambda i: (i, 0)),
            pl.BlockSpec((QB * K, H), lambda i: (i, 0)),
            pl.BlockSpec((QB * K, BT, IN_C), lambda i: (i, 0, 0)),
            pl.BlockSpec((QB, QB * K), lambda i: (0, 0)),
            pl.BlockSpec(B.shape, lambda i: (0, 0)),
            pl.BlockSpec(W_l1.shape, lambda i: (0, 0)),
            pl.BlockSpec(b_l1.shape, lambda i: (0,)),
            pl.BlockSpec((OUT_C, IN_C, H), lambda i: (0, 0, 0)),
            pl.BlockSpec((OUT_C, IN_C), lambda i: (0, 0)),
        ],
        out_specs=pl.BlockSpec((OUT_C, QB, BT), lambda i: (0, i, 0)),
    )(dist2, kd_e, qmw_e, qmo_e, xe, S, B, W_l1, b_l1, W_l2p, filt_p)

    out_flat = jnp.transpose(out_c, (2, 1, 0)).reshape(BT * NQ, OUT_C)
    out_final = pl.pallas_call(
        _epilogue_body,
        out_shape=jax.ShapeDtypeStruct((BT * NQ, OUT_C), jnp.float32),
        grid=(4,),
        in_specs=[
            pl.BlockSpec((BT * NQ // 4, OUT_C), lambda i: (i, 0)),
            pl.BlockSpec(bias_out.shape, lambda i: (0,)),
            pl.BlockSpec(W_m1.shape, lambda i: (0, 0)),
            pl.BlockSpec(b_m1.shape, lambda i: (0,)),
            pl.BlockSpec(W_m2.shape, lambda i: (0, 0)),
            pl.BlockSpec(b_m2.shape, lambda i: (0,)),
        ],
        out_specs=pl.BlockSpec((BT * NQ // 4, OUT_C), lambda i: (i, 0)),
    )(out_flat, bias_out, W_m1, b_m1, W_m2, b_m2)
    return (out_final.reshape(bsz, tsz, NQ, OUT_C), query_pos)
